# baseline (speedup 1.0000x reference)
import functools

import jax
import jax.numpy as jnp
from jax import lax
from jax.experimental import pallas as pl
from jax.experimental.pallas import tpu as pltpu
from jax.experimental.pallas import tpu_sc as plsc

BATCH = 16384
N_FIELDS = 4
DIM = 16
VOCAB = 1000000

_NC = 2
_NS = 16
_NW = _NC * _NS
_BPW = BATCH // _NW
_CHUNK = 128
_NCHUNK = _BPW // _CHUNK

_GRP = VOCAB // _CHUNK
_GPW = -(-_GRP // _NW)
_TAIL = VOCAB - _GRP * _CHUNK

_mesh = plsc.VectorSubcoreMesh(core_axis_name="c", subcore_axis_name="s")


@functools.partial(
    pl.kernel,
    mesh=_mesh,
    compiler_params=pltpu.CompilerParams(
        use_tc_tiling_on_sc=True, needs_layout_passes=False
    ),
    out_type=[jax.ShapeDtypeStruct((VOCAB * DIM,), jnp.float32)] * N_FIELDS,
    scratch_types=[
        pltpu.VMEM((2, DIM, _CHUNK), jnp.float32),
        pltpu.VMEM((DIM, _TAIL), jnp.float32),
        pltpu.VMEM((2, _CHUNK * DIM), jnp.float32),
        pltpu.SemaphoreType.DMA,
        pltpu.SemaphoreType.DMA,
    ],
)
def _untile(w0, w1, w2, w3, t0, t1, t2, t3, s0, s1, s2, s3,
            blk_v, tail_v, rows_v, semi, semo):
    wid = lax.axis_index("s") * _NC + lax.axis_index("c")
    iota = lax.iota(jnp.int32, 16)
    for wt, tt, st in zip([w0, w1, w2, w3], [t0, t1, t2, t3],
                          [s0, s1, s2, s3]):
        lo = wid * _GPW
        hi = jnp.minimum(lo + _GPW, _GRP)

        def fetch(g, slot, wt=wt):
            pltpu.make_async_copy(
                wt.at[:, pl.ds(pl.multiple_of(g * _CHUNK, 128), _CHUNK)],
                blk_v.at[slot],
                semi,
            ).start()

        fetch(lo, 0)

        @pl.loop(0, _GPW)
        def per_group(i, wt=wt, st=st, lo=lo, hi=hi, fetch=fetch):
            g = lo + i

            @pl.when(g < hi)
            def _():
                slot = lax.rem(i, 2)

                @pl.when(g + 1 < hi)
                def _():
                    fetch(g + 1, lax.rem(i + 1, 2))

                pltpu.make_async_copy(
                    wt.at[:, pl.ds(0, _CHUNK)], blk_v.at[0], semi
                ).wait()

                @pl.when(i >= 2)
                def _():
                    pltpu.make_async_copy(
                        st.at[pl.ds(0, _CHUNK * DIM)], rows_v.at[0], semo
                    ).wait()

                @plsc.parallel_loop(0, _CHUNK, unroll=16)
                def per_col(j, slot=slot):
                    col = plsc.load_gather(
                        blk_v.at[slot], [iota, jnp.full((16,), j, jnp.int32)]
                    )
                    rows_v[slot, pl.ds(j * DIM, DIM)] = col

                pltpu.make_async_copy(
                    rows_v.at[slot],
                    st.at[pl.ds(g * _CHUNK * DIM, _CHUNK * DIM)],
                    semo,
                ).start()

        nw = hi - lo
        for k in range(2):
            @pl.when(nw >= k + 1)
            def _(st=st):
                pltpu.make_async_copy(
                    st.at[pl.ds(0, _CHUNK * DIM)], rows_v.at[0], semo
                ).wait()

        @pl.when(wid == 0)
        def _(tt=tt, st=st):
            pltpu.sync_copy(tt, tail_v)

            @plsc.parallel_loop(0, _TAIL, unroll=16)
            def per_tail_col(j):
                col = plsc.load_gather(
                    tail_v, [iota, jnp.full((16,), j, jnp.int32)]
                )
                rows_v[0, pl.ds(j * DIM, DIM)] = col

            pltpu.sync_copy(
                rows_v.at[0, pl.ds(0, _TAIL * DIM)],
                st.at[pl.ds(_GRP * _CHUNK * DIM, _TAIL * DIM)],
            )


@functools.partial(
    pl.kernel,
    mesh=_mesh,
    compiler_params=pltpu.CompilerParams(use_tc_tiling_on_sc=False),
    out_type=jax.ShapeDtypeStruct((BATCH, N_FIELDS, DIM), jnp.float32),
    scratch_types=[
        pltpu.VMEM((N_FIELDS, _BPW), jnp.int32),
        pltpu.VMEM((N_FIELDS, _BPW, DIM), jnp.float32),
        pltpu.SemaphoreType.DMA,
    ],
)
def _emb_lookup(xT_hbm, w0, w1, w2, w3, out_hbm, idx_v, rows_v, sem):
    wid = lax.axis_index("s") * _NC + lax.axis_index("c")
    base = wid * _BPW
    pltpu.sync_copy(xT_hbm.at[:, pl.ds(base, _BPW)], idx_v)
    tables = [w0, w1, w2, w3]
    handles = []
    for f in range(N_FIELDS):
        for c in range(_NCHUNK):
            handles.append(
                pltpu.async_copy(
                    tables[f].at[idx_v.at[f, pl.ds(c * _CHUNK, _CHUNK)]],
                    rows_v.at[f, pl.ds(c * _CHUNK, _CHUNK)],
                    sem,
                )
            )
    for h in handles:
        h.wait()
    for f in range(N_FIELDS):
        pltpu.sync_copy(rows_v.at[f], out_hbm.at[pl.ds(base, _BPW), f])


def kernel(x, W0, W1, W2, W3):
    xT = x.astype(jnp.int32).T
    tails = [W.T[:, _GRP * _CHUNK:] for W in (W0, W1, W2, W3)]
    s0, s1, s2, s3 = _untile(W0.T, W1.T, W2.T, W3.T, *tails)
    out = _emb_lookup(
        xT,
        s0.reshape(VOCAB, DIM),
        s1.reshape(VOCAB, DIM),
        s2.reshape(VOCAB, DIM),
        s3.reshape(VOCAB, DIM),
    )
    return out.reshape(BATCH, N_FIELDS * DIM)


# speedup vs baseline: 1.0017x; 1.0017x over previous
import functools

import jax
import jax.numpy as jnp
from jax import lax
from jax.experimental import pallas as pl
from jax.experimental.pallas import tpu as pltpu
from jax.experimental.pallas import tpu_sc as plsc

BATCH = 16384
N_FIELDS = 4
DIM = 16
VOCAB = 1000000

_NC = 2
_NS = 16
_NW = _NC * _NS
_BPW = BATCH // _NW
_CHUNK = 128
_NCHUNK = _BPW // _CHUNK

_GRP = VOCAB // _CHUNK
_GPW = -(-_GRP // _NW)
_TAIL = VOCAB - _GRP * _CHUNK

_mesh = plsc.VectorSubcoreMesh(core_axis_name="c", subcore_axis_name="s")


@functools.partial(
    pl.kernel,
    mesh=_mesh,
    compiler_params=pltpu.CompilerParams(
        use_tc_tiling_on_sc=True, needs_layout_passes=False
    ),
    out_type=[jax.ShapeDtypeStruct((VOCAB * DIM,), jnp.float32)] * N_FIELDS,
    scratch_types=[
        pltpu.VMEM((2, DIM, _CHUNK), jnp.float32),
        pltpu.VMEM((DIM, _TAIL), jnp.float32),
        pltpu.VMEM((2, _CHUNK * DIM), jnp.float32),
        pltpu.SemaphoreType.DMA,
        pltpu.SemaphoreType.DMA,
    ],
)
def _untile(w0, w1, w2, w3, t0, t1, t2, t3, s0, s1, s2, s3,
            blk_v, tail_v, rows_v, semi, semo):
    wid = lax.axis_index("s") * _NC + lax.axis_index("c")
    iota = lax.iota(jnp.int32, 16)
    for wt, tt, st in zip([w0, w1, w2, w3], [t0, t1, t2, t3],
                          [s0, s1, s2, s3]):
        lo = wid * _GPW
        hi = jnp.minimum(lo + _GPW, _GRP)

        def fetch(g, slot, wt=wt):
            pltpu.make_async_copy(
                wt.at[:, pl.ds(pl.multiple_of(g * _CHUNK, 128), _CHUNK)],
                blk_v.at[slot],
                semi,
            ).start()

        fetch(lo, 0)

        @pl.loop(0, _GPW)
        def per_group(i, wt=wt, st=st, lo=lo, hi=hi, fetch=fetch):
            g = lo + i

            @pl.when(g < hi)
            def _():
                slot = lax.rem(i, 2)

                @pl.when(g + 1 < hi)
                def _():
                    fetch(g + 1, lax.rem(i + 1, 2))

                pltpu.make_async_copy(
                    wt.at[:, pl.ds(0, _CHUNK)], blk_v.at[0], semi
                ).wait()

                @pl.when(i >= 2)
                def _():
                    pltpu.make_async_copy(
                        st.at[pl.ds(0, _CHUNK * DIM)], rows_v.at[0], semo
                    ).wait()

                @plsc.parallel_loop(0, _CHUNK // 4, unroll=4)
                def per_col(j, slot=slot):
                    for q in range(4):
                        jj = j + q * (_CHUNK // 4)
                        col = plsc.load_gather(
                            blk_v.at[slot],
                            [iota, jnp.full((16,), jj, jnp.int32)],
                        )
                        rows_v[slot, pl.ds(jj * DIM, DIM)] = col

                pltpu.make_async_copy(
                    rows_v.at[slot],
                    st.at[pl.ds(g * _CHUNK * DIM, _CHUNK * DIM)],
                    semo,
                ).start()

        nw = hi - lo
        for k in range(2):
            @pl.when(nw >= k + 1)
            def _(st=st):
                pltpu.make_async_copy(
                    st.at[pl.ds(0, _CHUNK * DIM)], rows_v.at[0], semo
                ).wait()

        @pl.when(wid == 0)
        def _(tt=tt, st=st):
            pltpu.sync_copy(tt, tail_v)

            @plsc.parallel_loop(0, _TAIL, unroll=16)
            def per_tail_col(j):
                col = plsc.load_gather(
                    tail_v, [iota, jnp.full((16,), j, jnp.int32)]
                )
                rows_v[0, pl.ds(j * DIM, DIM)] = col

            pltpu.sync_copy(
                rows_v.at[0, pl.ds(0, _TAIL * DIM)],
                st.at[pl.ds(_GRP * _CHUNK * DIM, _TAIL * DIM)],
            )


@functools.partial(
    pl.kernel,
    mesh=_mesh,
    compiler_params=pltpu.CompilerParams(use_tc_tiling_on_sc=False),
    out_type=jax.ShapeDtypeStruct((BATCH, N_FIELDS, DIM), jnp.float32),
    scratch_types=[
        pltpu.VMEM((N_FIELDS, _BPW), jnp.int32),
        pltpu.VMEM((N_FIELDS, _BPW, DIM), jnp.float32),
        pltpu.SemaphoreType.DMA,
    ],
)
def _emb_lookup(xT_hbm, w0, w1, w2, w3, out_hbm, idx_v, rows_v, sem):
    wid = lax.axis_index("s") * _NC + lax.axis_index("c")
    base = wid * _BPW
    pltpu.sync_copy(xT_hbm.at[:, pl.ds(base, _BPW)], idx_v)
    tables = [w0, w1, w2, w3]
    handles = []
    for f in range(N_FIELDS):
        for c in range(_NCHUNK):
            handles.append(
                pltpu.async_copy(
                    tables[f].at[idx_v.at[f, pl.ds(c * _CHUNK, _CHUNK)]],
                    rows_v.at[f, pl.ds(c * _CHUNK, _CHUNK)],
                    sem,
                )
            )
    for h in handles:
        h.wait()
    for f in range(N_FIELDS):
        pltpu.sync_copy(rows_v.at[f], out_hbm.at[pl.ds(base, _BPW), f])


def kernel(x, W0, W1, W2, W3):
    xT = x.astype(jnp.int32).T
    tails = [W.T[:, _GRP * _CHUNK:] for W in (W0, W1, W2, W3)]
    s0, s1, s2, s3 = _untile(W0.T, W1.T, W2.T, W3.T, *tails)
    out = _emb_lookup(
        xT,
        s0.reshape(VOCAB, DIM),
        s1.reshape(VOCAB, DIM),
        s2.reshape(VOCAB, DIM),
        s3.reshape(VOCAB, DIM),
    )
    return out.reshape(BATCH, N_FIELDS * DIM)
